# Initial kernel scaffold; baseline (speedup 1.0000x reference)
import jax
import jax.numpy as jnp
from jax.experimental import pallas as pl


def kernel(indices, weights):
    raise NotImplementedError("write your pallas kernel here")


# speedup vs baseline: 1.5625x; 1.5625x over previous
import functools

import jax
import jax.numpy as jnp
from jax import lax
from jax.experimental import pallas as pl
from jax.experimental.pallas import tpu as pltpu
from jax.experimental.pallas import tpu_sc as plsc


def _gather_fn(B, D, CH, num_ch, b_per_w, num_cores):
    mesh = plsc.VectorSubcoreMesh(core_axis_name="c", subcore_axis_name="s")

    @functools.partial(
        pl.kernel,
        mesh=mesh,
        out_type=jax.ShapeDtypeStruct((B, D), jnp.float32),
        scratch_types=[
            pltpu.VMEM((CH,), jnp.int32),
            pltpu.VMEM((CH, D), jnp.float32),
            pltpu.SemaphoreType.DMA,
        ],
        compiler_params=pltpu.CompilerParams(use_tc_tiling_on_sc=False),
    )
    def k(idx_hbm, table_hbm, out_hbm, idx_v, rows_v, sem):
        wid = lax.axis_index("s") * num_cores + lax.axis_index("c")
        base = wid * b_per_w

        def body(i, carry):
            off = base + i * CH
            pltpu.sync_copy(idx_hbm.at[pl.ds(off, CH)], idx_v)
            pltpu.async_copy(table_hbm.at[idx_v], rows_v, sem).wait()
            pltpu.sync_copy(rows_v, out_hbm.at[pl.ds(off, CH)])
            return carry

        lax.fori_loop(0, num_ch, body, 0)

    return k


def kernel(indices, weights):
    Bt, F = indices.shape
    V, D = weights.shape
    B = Bt * F
    idx_flat = indices.reshape(B).astype(jnp.int32)

    info = plsc.get_sparse_core_info()
    NW = info.num_cores * info.num_subcores
    b_per_w = B // NW
    CH = 1664
    num_ch = b_per_w // CH
    assert b_per_w % CH == 0 and B % NW == 0

    out = _gather_fn(B, D, CH, num_ch, b_per_w, info.num_cores)(idx_flat, weights)
    return out.reshape(Bt, F, D)


# speedup vs baseline: 1.5760x; 1.0086x over previous
import functools

import jax
import jax.numpy as jnp
from jax import lax
from jax.experimental import pallas as pl
from jax.experimental.pallas import tpu as pltpu
from jax.experimental.pallas import tpu_sc as plsc

_NBUF = 2


def _gather_fn(B, D, CH, num_ch, b_per_w, num_cores):
    mesh = plsc.VectorSubcoreMesh(core_axis_name="c", subcore_axis_name="s")

    @functools.partial(
        pl.kernel,
        mesh=mesh,
        out_type=jax.ShapeDtypeStruct((B, D), jnp.float32),
        scratch_types=[
            pltpu.VMEM((b_per_w,), jnp.int32),
            [pltpu.VMEM((CH, D), jnp.float32) for _ in range(_NBUF)],
            [pltpu.SemaphoreType.DMA for _ in range(_NBUF)],
            [pltpu.SemaphoreType.DMA for _ in range(_NBUF)],
        ],
        compiler_params=pltpu.CompilerParams(use_tc_tiling_on_sc=False),
    )
    def k(idx_hbm, table_hbm, out_hbm, idx_v, rows, g_sems, s_sems):
        wid = lax.axis_index("s") * num_cores + lax.axis_index("c")
        base = wid * b_per_w
        pltpu.sync_copy(idx_hbm.at[pl.ds(base, b_per_w)], idx_v)

        gathers = [None] * num_ch
        stores = [None] * num_ch

        def fire_gather(i):
            s = i % _NBUF
            gathers[i] = pltpu.async_copy(
                table_hbm.at[idx_v.at[pl.ds(i * CH, CH)]], rows[s], g_sems[s]
            )

        for b in range(min(_NBUF, num_ch)):
            fire_gather(b)
        for i in range(num_ch):
            s = i % _NBUF
            gathers[i].wait()
            stores[i] = pltpu.async_copy(
                rows[s], out_hbm.at[pl.ds(base + i * CH, CH)], s_sems[s]
            )
            nxt = i + _NBUF
            if nxt < num_ch:
                stores[i].wait()
                fire_gather(nxt)
        for i in range(max(0, num_ch - _NBUF), num_ch):
            stores[i].wait()

    return k


def kernel(indices, weights):
    Bt, F = indices.shape
    V, D = weights.shape
    B = Bt * F
    idx_flat = indices.reshape(B).astype(jnp.int32)

    info = plsc.get_sparse_core_info()
    NW = info.num_cores * info.num_subcores
    b_per_w = B // NW
    CH = 1664
    num_ch = b_per_w // CH
    assert b_per_w % CH == 0 and B % NW == 0

    out = _gather_fn(B, D, CH, num_ch, b_per_w, info.num_cores)(idx_flat, weights)
    return out.reshape(Bt, F, D)
